# baseline (speedup 1.0000x reference)
import functools

import jax
import jax.numpy as jnp
from jax import lax
from jax.experimental import pallas as pl
from jax.experimental.pallas import tpu as pltpu
from jax.experimental.pallas import tpu_sc as plsc

BATCH = 4096
SEQ = 200
EMB_D = 32
TOTAL = BATCH * SEQ

_info = plsc.get_sparse_core_info()
_NC, _NS = _info.num_cores, _info.num_subcores
NW = _NC * _NS
B_PER_W = TOTAL // NW
NSLOT = 4
CHUNK = 640
NCHUNK = B_PER_W // CHUNK
NROUND = NCHUNK // NSLOT

_mesh = plsc.VectorSubcoreMesh(core_axis_name="c", subcore_axis_name="s")


@functools.partial(
    pl.kernel,
    mesh=_mesh,
    out_type=jax.ShapeDtypeStruct((TOTAL, EMB_D), jnp.float32),
    scratch_types=(
        [pltpu.VMEM((B_PER_W,), jnp.int32),
         pltpu.VMEM((NSLOT, CHUNK, EMB_D), jnp.float32)]
        + [pltpu.SemaphoreType.DMA] * (2 * NSLOT)
    ),
    compiler_params=pltpu.CompilerParams(use_tc_tiling_on_sc=False),
)
def _emb_lookup(idx_hbm, table_hbm, out_hbm, idx_v, rows_v, *sems):
    gsems, wsems = sems[:NSLOT], sems[NSLOT:]
    wid = lax.axis_index("s") * _NC + lax.axis_index("c")
    base = wid * B_PER_W

    pltpu.sync_copy(idx_hbm.at[pl.ds(base, B_PER_W)], idx_v)

    @pl.loop(0, NROUND)
    def _round(j):
        gathers = []
        for b in range(NSLOT):
            @pl.when(j > 0)
            def _drain_prev_write():
                pltpu.make_async_copy(
                    rows_v.at[b], out_hbm.at[pl.ds(base, CHUNK)], wsems[b]
                ).wait()

            chunk_ids = idx_v.at[pl.ds((j * NSLOT + b) * CHUNK, CHUNK)]
            gathers.append(
                pltpu.async_copy(table_hbm.at[chunk_ids], rows_v.at[b], gsems[b])
            )
        for b in range(NSLOT):
            off = base + (j * NSLOT + b) * CHUNK
            gathers[b].wait()
            pltpu.async_copy(rows_v.at[b], out_hbm.at[pl.ds(off, CHUNK)], wsems[b])

    for b in range(NSLOT):
        pltpu.make_async_copy(
            rows_v.at[b], out_hbm.at[pl.ds(base, CHUNK)], wsems[b]
        ).wait()


def kernel(token_ids, weight):
    flat = token_ids.reshape(-1).astype(jnp.int32)
    out = _emb_lookup(flat, weight)
    return out.reshape(BATCH, SEQ, EMB_D)


# speedup vs baseline: 1.0050x; 1.0050x over previous
import functools

import jax
import jax.numpy as jnp
from jax import lax
from jax.experimental import pallas as pl
from jax.experimental.pallas import tpu as pltpu
from jax.experimental.pallas import tpu_sc as plsc

BATCH = 4096
SEQ = 200
EMB_D = 32
TOTAL = BATCH * SEQ

_info = plsc.get_sparse_core_info()
_NC, _NS = _info.num_cores, _info.num_subcores
NW = _NC * _NS
B_PER_W = TOTAL // NW
NSLOT = 4
CHUNK = 800
NCHUNK = B_PER_W // CHUNK
NROUND = NCHUNK // NSLOT

_mesh = plsc.VectorSubcoreMesh(core_axis_name="c", subcore_axis_name="s")


@functools.partial(
    pl.kernel,
    mesh=_mesh,
    out_type=jax.ShapeDtypeStruct((TOTAL, EMB_D), jnp.float32),
    scratch_types=(
        [pltpu.VMEM((NSLOT, CHUNK), jnp.int32),
         pltpu.VMEM((NSLOT, CHUNK, EMB_D), jnp.float32)]
        + [pltpu.SemaphoreType.DMA] * (2 * NSLOT)
    ),
    compiler_params=pltpu.CompilerParams(use_tc_tiling_on_sc=False),
)
def _emb_lookup(idx_hbm, table_hbm, out_hbm, idx_v, rows_v, *sems):
    gsems, wsems = sems[:NSLOT], sems[NSLOT:]
    wid = lax.axis_index("s") * _NC + lax.axis_index("c")
    base = wid * B_PER_W

    @pl.loop(0, NROUND)
    def _round(j):
        gathers = []
        for b in range(NSLOT):
            off = base + (j * NSLOT + b) * CHUNK

            @pl.when(j > 0)
            def _drain_prev_write():
                pltpu.make_async_copy(
                    rows_v.at[b], out_hbm.at[pl.ds(base, CHUNK)], wsems[b]
                ).wait()

            pltpu.sync_copy(idx_hbm.at[pl.ds(off, CHUNK)], idx_v.at[b])
            gathers.append(
                pltpu.async_copy(table_hbm.at[idx_v.at[b]], rows_v.at[b], gsems[b])
            )
        for b in range(NSLOT):
            off = base + (j * NSLOT + b) * CHUNK
            gathers[b].wait()
            pltpu.async_copy(rows_v.at[b], out_hbm.at[pl.ds(off, CHUNK)], wsems[b])

    for b in range(NSLOT):
        pltpu.make_async_copy(
            rows_v.at[b], out_hbm.at[pl.ds(base, CHUNK)], wsems[b]
        ).wait()


def kernel(token_ids, weight):
    flat = token_ids.reshape(-1).astype(jnp.int32)
    out = _emb_lookup(flat, weight)
    return out.reshape(BATCH, SEQ, EMB_D)
